# baseline (speedup 1.0000x reference)
import functools
import math as _math

import jax
import jax.numpy as jnp
import numpy as _np
from jax import lax
from jax.experimental import pallas as pl
from jax.experimental.pallas import tpu as pltpu
from jax.experimental.pallas import tpu_sc as plsc

NC = 2
NS = 16
NW = NC * NS
CH = 128


def _wid():
    return lax.axis_index("s") * NC + lax.axis_index("c")


def _sc_mesh():
    return plsc.VectorSubcoreMesh(core_axis_name="c", subcore_axis_name="s")


def _build_hq_table(rela_embed, q_rel_pad):
    bp = q_rel_pad.shape[0]
    d = rela_embed.shape[1]
    per = bp // NW
    c0 = 64
    nck = per // c0

    @functools.partial(
        pl.kernel,
        mesh=_sc_mesh(),
        out_type=jax.ShapeDtypeStruct((bp, d), jnp.float32),
        scratch_types=[
            pltpu.VMEM((c0,), jnp.int32),
            pltpu.VMEM((c0, d), jnp.float32),
            pltpu.SemaphoreType.DMA,
        ],
    )
    def k(q_h, tab_h, out_h, idx_v, rows_v, sem):
        base = _wid() * per

        def body(j, carry):
            off = pl.multiple_of(base + j * c0, 8)
            pltpu.sync_copy(q_h.at[pl.ds(off, c0)], idx_v)
            pltpu.async_copy(tab_h.at[idx_v], rows_v, sem).wait()
            pltpu.sync_copy(rows_v, out_h.at[pl.ds(off, c0)])
            return carry

        lax.fori_loop(0, nck, body, 0)

    return k(q_rel_pad, rela_embed)


def _gather_staged(tt4, idx_st, n_tab, rows_per_tab):
    total = idx_st.shape[0]
    d = tt4.shape[1]
    ep = total // n_tab
    per_t = ep // CH // NW
    G = next(g for g in (20, 16, 10, 8, 5, 4, 2, 1) if per_t % g == 0)
    R = 2
    ngr = per_t // G
    srows = rows_per_tab // NS

    @functools.partial(
        pl.kernel,
        mesh=_sc_mesh(),
        out_type=jax.ShapeDtypeStruct((total, d), jnp.float32),
        scratch_types=[
            pltpu.VMEM_SHARED((rows_per_tab, d), jnp.float32),
            pltpu.VMEM((G * CH,), jnp.int32),
            [pltpu.VMEM((CH, d), jnp.float32)] * R,
            [pltpu.SemaphoreType.DMA] * R,
            [pltpu.SemaphoreType.DMA] * R,
        ],
    )
    def k(tt_h, idx_h, out_h, stab, idxb, bufs, gsem, ssem):
        sid = lax.axis_index("s")
        base = _wid() * per_t * CH
        ssl = pl.ds(sid * srows, srows)

        for t in range(n_tab):
            pltpu.sync_copy(tt_h.at[pl.ds(t * rows_per_tab + sid * srows,
                                          srows)], stab.at[ssl])
            plsc.subcore_barrier()

            def group(g, carry):
                goff = pl.multiple_of(t * ep + base + g * (G * CH), 8)
                pltpu.sync_copy(idx_h.at[pl.ds(goff, G * CH)], idxb)
                pltpu.async_copy(stab.at[idxb.at[pl.ds(0, CH)]],
                                 bufs[0], gsem[0])
                for c in range(G):
                    r = c % R
                    if c + 1 < G:
                        rr = (c + 1) % R
                        if c - 1 >= 0:
                            pltpu.make_async_copy(
                                bufs[rr],
                                out_h.at[pl.ds(goff + (c - 1) * CH, CH)],
                                ssem[rr]).wait()
                        pltpu.async_copy(
                            stab.at[idxb.at[pl.ds((c + 1) * CH, CH)]],
                            bufs[rr], gsem[rr])
                    pltpu.make_async_copy(stab.at[idxb.at[pl.ds(c * CH, CH)]],
                                          bufs[r], gsem[r]).wait()
                    pltpu.async_copy(bufs[r],
                                     out_h.at[pl.ds(goff + c * CH, CH)],
                                     ssem[r])
                for r in range(R):
                    pltpu.make_async_copy(bufs[r], out_h.at[pl.ds(goff, CH)],
                                          ssem[r]).wait()
                return carry

            lax.fori_loop(0, ngr, group, 0)
            plsc.subcore_barrier()

    return k(tt4, idx_st)


def _scatter_segments(up, ee, obj, np_rows, z_up):
    ep = obj.shape[0]
    per = ep // NW
    nck = per // CH
    d = up.shape[1]
    zrows = np_rows // NS

    @functools.partial(
        pl.kernel,
        mesh=_sc_mesh(),
        out_type=(
            jax.ShapeDtypeStruct((NC, np_rows, d), jnp.float32),
            jax.ShapeDtypeStruct((NC, np_rows, d), jnp.float32),
        ),
        scratch_types=[
            pltpu.VMEM((CH,), jnp.int32),
            pltpu.VMEM((CH, d), jnp.float32),
            pltpu.VMEM_SHARED((np_rows, d), jnp.float32),
        ],
    )
    def k(up_h, ee_h, obj_h, zu_h, pu_o, pe_o, idx_v, bu, acc_u):
        sid = lax.axis_index("s")
        cid = lax.axis_index("c")
        base = (sid * NC + cid) * per
        zsl = pl.ds(sid * zrows, zrows)

        for src_h, out_o in ((up_h, pu_o), (ee_h, pe_o)):
            pltpu.sync_copy(zu_h, acc_u.at[zsl])
            plsc.subcore_barrier()

            def body(j, carry):
                off = pl.multiple_of(base + j * CH, 8)
                sl = pl.ds(off, CH)
                pltpu.sync_copy(obj_h.at[sl], idx_v)
                pltpu.sync_copy(src_h.at[sl], bu)
                pltpu.sync_copy(bu, acc_u.at[idx_v], add=True)
                return carry

            lax.fori_loop(0, nck, body, 0)
            plsc.subcore_barrier()
            pltpu.sync_copy(acc_u.at[zsl], out_o.at[cid, zsl])
            plsc.subcore_barrier()

    return k(up, ee, obj, z_up)


def _lrelu(x):
    return jnp.maximum(x, 0.01 * x)


def _edge_body(x_r, w1a_r, w1b_r, b1_r, w2_r, b2_r,
               g1_r, g2_r, g3_r, gb_r, h1_r, h2_r, hb_r,
               wst_r, wrt_r, wqrt_r, qb_r, wa_r, up_r, ee_r):
    dot = lambda a, b: jnp.dot(a.astype(jnp.bfloat16), b,
                               preferred_element_type=jnp.float32)
    x = x_r[...]
    blk = x.shape[1] * x.shape[2]
    d = x.shape[3]
    hs = x[0].reshape(blk, d)
    rel = x[1].reshape(blk, d)
    tpe = x[2].reshape(blk, d)
    hq = x[3].reshape(blk, d)
    h1 = _lrelu(dot(rel, w1a_r[...]) + dot(tpe, w1b_r[...]) + b1_r[...])
    h2 = _lrelu(dot(h1, w2_r[...]) + b2_r[...])
    hr = h2 + rel
    g = jax.nn.sigmoid(dot(hr, g1_r[...]) + dot(hq, g2_r[...])
                       + dot(hs, g3_r[...]) + gb_r[...])
    d = hs.shape[1]
    upd = g[:, :d]
    rst = g[:, d:]
    cand = jnp.tanh(dot(hr, h1_r[...]) + dot(rst * hs, h2_r[...]) + hb_r[...])
    msg = (1.0 - upd) * hs + upd * cand
    al = _lrelu(dot(hs, wst_r[...]) + dot(hr, wrt_r[...])
                + dot(hq, wqrt_r[...]) + qb_r[...])
    a = jnp.sum(al * wa_r[...], axis=1, keepdims=True)
    e = jnp.exp(a)
    up_r[...] = e * msg
    ee_r[...] = jnp.broadcast_to(e, (e.shape[0], ee_r.shape[1]))


def _edge_compute(x4, w1a, w1b, b1, w2, b2,
                  g1, g2, g3, gb, h1w, h2w, hb, wst, wrt, wqrt, qb, wa):
    nst, nch, ch, d = x4.shape
    blk = 2048
    cpb = blk // ch
    grid = nch // cpb
    ep = nch * ch
    row = lambda n: pl.BlockSpec((blk, n), lambda i: (i, 0))
    full = lambda a: pl.BlockSpec(a.shape, lambda i: (0,) * a.ndim)
    return pl.pallas_call(
        _edge_body,
        grid=(grid,),
        in_specs=[pl.BlockSpec((nst, cpb, ch, d), lambda i: (0, i, 0, 0))]
                 + [full(x) for x in (w1a, w1b, b1, w2, b2, g1, g2, g3,
                                      gb, h1w, h2w, hb, wst, wrt, wqrt, qb, wa)],
        out_specs=(row(d), row(d)),
        out_shape=(jax.ShapeDtypeStruct((ep, d), jnp.float32),
                   jax.ShapeDtypeStruct((ep, d), jnp.float32)),
    )(x4, w1a, w1b, b1, w2, b2, g1, g2, g3, gb,
      h1w, h2w, hb, wst, wrt, wqrt, qb, wa)


def _finish_body(pu_r, pe_r, wh_r, out_r):
    pu = pu_r[...]
    pe = pe_r[...]
    s = pu[0] + pu[1]
    b = pe[0, :, :1] + pe[1, :, :1] + 1e-5
    out_r[...] = jnp.dot(s / b, wh_r[...], preferred_element_type=jnp.float32)


def _finish(pu, pe, wh_t, n):
    d = pu.shape[2]
    de = pe.shape[2]
    blk = 2000
    grid = n // blk
    return pl.pallas_call(
        _finish_body,
        grid=(grid,),
        in_specs=[pl.BlockSpec((NC, blk, d), lambda i: (0, i, 0)),
                  pl.BlockSpec((NC, blk, de), lambda i: (0, i, 0)),
                  pl.BlockSpec(wh_t.shape, lambda i: (0, 0))],
        out_specs=pl.BlockSpec((blk, d), lambda i: (i, 0)),
        out_shape=jax.ShapeDtypeStruct((n, d), jnp.float32),
    )(pu, pe, wh_t)


def kernel(q_sub, q_rel, hidden, edges, n_node, rela_embed, time_pe,
           Ws_W, Wr_W, Wqr_W, Wqr_b, fuse_W1, fuse_b1, fuse_W2, fuse_b2,
           wA, gate_gW, gate_gb, gate_hW, gate_hb, Wh):
    e = edges.shape[0]
    n = hidden.shape[0]
    d = hidden.shape[1]

    step = NW * CH
    ep = ((e + step - 1) // step) * step
    pad = ep - e
    r0 = jnp.concatenate([edges[:, 0], jnp.zeros((pad,), jnp.int32)])
    r2 = jnp.concatenate([edges[:, 2], jnp.zeros((pad,), jnp.int32)])
    sub = jnp.concatenate([edges[:, 4], jnp.zeros((pad,), jnp.int32)])
    t6 = jnp.concatenate([edges[:, 6], jnp.zeros((pad,), jnp.int32)])

    b = q_rel.shape[0]
    bstep = 64 * NW
    bp = ((b + bstep - 1) // bstep) * bstep
    q_rel_pad = jnp.concatenate([q_rel, jnp.zeros((bp - b,), jnp.int32)])

    np_rows = ((n + 1 + NS * 8 - 1) // (NS * 8)) * (NS * 8)
    obj = jnp.concatenate([edges[:, 5], jnp.full((pad,), n, jnp.int32)])

    hq_table = _build_hq_table(rela_embed, q_rel_pad)

    dt = time_pe.shape[1]
    time_pe_pad = jnp.pad(time_pe, ((0, 0), (0, d - dt)))
    nt_rows = bp
    padr = lambda a: jnp.pad(a, ((0, nt_rows - a.shape[0]), (0, 0)))
    tt4 = jnp.concatenate(
        [padr(hidden), padr(rela_embed), padr(time_pe_pad), hq_table], axis=0)
    idx_st = jnp.concatenate([sub, r2, t6, r0])
    out_rows = _gather_staged(tt4, idx_st, 4, nt_rows)
    nchk = ep // CH
    x4 = out_rows.reshape(4, nchk, CH, d)

    bf = lambda a: a.astype(jnp.bfloat16)
    w1a = fuse_W1[:, :d].T
    w1b = jnp.pad(fuse_W1[:, d:].T, ((0, d - dt), (0, 0)))
    gt = gate_gW.T
    ht = gate_hW.T
    up, ee = _edge_compute(
        x4,
        bf(w1a), bf(w1b), fuse_b1.reshape(1, -1), bf(fuse_W2.T),
        fuse_b2.reshape(1, -1),
        bf(gt[:d]), bf(gt[d:2 * d]), bf(gt[2 * d:]), gate_gb.reshape(1, -1),
        bf(ht[:d]), bf(ht[d:]), gate_hb.reshape(1, -1),
        bf(Ws_W.T), bf(Wr_W.T), bf(Wqr_W.T), Wqr_b.reshape(1, -1), wA)

    zrows = np_rows // NS
    z_up = jnp.zeros((zrows, d), jnp.float32)
    pu, pe = _scatter_segments(up, ee, obj, np_rows, z_up)
    return _finish(pu, pe, Wh.T, n)


# speedup vs baseline: 1.1494x; 1.1494x over previous
import functools
import math as _math

import jax
import jax.numpy as jnp
import numpy as _np
from jax import lax
from jax.experimental import pallas as pl
from jax.experimental.pallas import tpu as pltpu
from jax.experimental.pallas import tpu_sc as plsc

NC = 2
NS = 16
NW = NC * NS
CH = 128


def _wid():
    return lax.axis_index("s") * NC + lax.axis_index("c")


def _sc_mesh():
    return plsc.VectorSubcoreMesh(core_axis_name="c", subcore_axis_name="s")


def _build_hq_table(rela_embed, q_rel_pad):
    bp = q_rel_pad.shape[0]
    d = rela_embed.shape[1]
    per = bp // NW
    c0 = 64
    nck = per // c0

    @functools.partial(
        pl.kernel,
        mesh=_sc_mesh(),
        out_type=jax.ShapeDtypeStruct((bp, d), jnp.float32),
        scratch_types=[
            pltpu.VMEM((c0,), jnp.int32),
            pltpu.VMEM((c0, d), jnp.float32),
            pltpu.SemaphoreType.DMA,
        ],
    )
    def k(q_h, tab_h, out_h, idx_v, rows_v, sem):
        base = _wid() * per

        def body(j, carry):
            off = pl.multiple_of(base + j * c0, 8)
            pltpu.sync_copy(q_h.at[pl.ds(off, c0)], idx_v)
            pltpu.async_copy(tab_h.at[idx_v], rows_v, sem).wait()
            pltpu.sync_copy(rows_v, out_h.at[pl.ds(off, c0)])
            return carry

        lax.fori_loop(0, nck, body, 0)

    return k(q_rel_pad, rela_embed)


def _gather_staged(tt4, idx_st, n_tab, rows_per_tab):
    total = idx_st.shape[0]
    d = tt4.shape[1]
    ep = total // n_tab
    per_t = ep // CH // NW
    G = next(g for g in (20, 16, 10, 8, 5, 4, 2, 1) if per_t % g == 0)
    R = 2
    ngr = per_t // G
    srows = rows_per_tab // NS

    @functools.partial(
        pl.kernel,
        mesh=_sc_mesh(),
        out_type=jax.ShapeDtypeStruct((total, d), jnp.float32),
        scratch_types=[
            pltpu.VMEM_SHARED((rows_per_tab, d), jnp.float32),
            pltpu.VMEM((G * CH,), jnp.int32),
            [pltpu.VMEM((CH, d), jnp.float32)] * R,
            [pltpu.SemaphoreType.DMA] * R,
            [pltpu.SemaphoreType.DMA] * R,
        ],
    )
    def k(tt_h, idx_h, out_h, stab, idxb, bufs, gsem, ssem):
        sid = lax.axis_index("s")
        base = _wid() * per_t * CH
        ssl = pl.ds(sid * srows, srows)

        for t in range(n_tab):
            pltpu.sync_copy(tt_h.at[pl.ds(t * rows_per_tab + sid * srows,
                                          srows)], stab.at[ssl])
            plsc.subcore_barrier()

            def group(g, carry):
                goff = pl.multiple_of(t * ep + base + g * (G * CH), 8)
                pltpu.sync_copy(idx_h.at[pl.ds(goff, G * CH)], idxb)
                pltpu.async_copy(stab.at[idxb.at[pl.ds(0, CH)]],
                                 bufs[0], gsem[0])
                for c in range(G):
                    r = c % R
                    if c + 1 < G:
                        rr = (c + 1) % R
                        if c - 1 >= 0:
                            pltpu.make_async_copy(
                                bufs[rr],
                                out_h.at[pl.ds(goff + (c - 1) * CH, CH)],
                                ssem[rr]).wait()
                        pltpu.async_copy(
                            stab.at[idxb.at[pl.ds((c + 1) * CH, CH)]],
                            bufs[rr], gsem[rr])
                    pltpu.make_async_copy(stab.at[idxb.at[pl.ds(c * CH, CH)]],
                                          bufs[r], gsem[r]).wait()
                    pltpu.async_copy(bufs[r],
                                     out_h.at[pl.ds(goff + c * CH, CH)],
                                     ssem[r])
                for r in range(R):
                    pltpu.make_async_copy(bufs[r], out_h.at[pl.ds(goff, CH)],
                                          ssem[r]).wait()
                return carry

            lax.fori_loop(0, ngr, group, 0)
            plsc.subcore_barrier()

    return k(tt4, idx_st)


def _scatter_segments(up, ee, obj2, np_rows, z_up):
    nchk = obj2.shape[0]
    ep = nchk * CH
    per = ep // NW
    nck = per // CH
    d = up.shape[1]
    zrows = np_rows // NS

    @functools.partial(
        pl.kernel,
        mesh=_sc_mesh(),
        out_type=(
            jax.ShapeDtypeStruct((NC, np_rows, d), jnp.float32),
            jax.ShapeDtypeStruct((NC, np_rows, d), jnp.float32),
        ),
        scratch_types=[
            pltpu.VMEM((nck, CH), jnp.int32),
            [pltpu.VMEM((CH, d), jnp.float32)] * 2,
            pltpu.VMEM_SHARED((np_rows, d), jnp.float32),
            [pltpu.SemaphoreType.DMA] * 2,
            [pltpu.SemaphoreType.DMA] * 2,
        ],
    )
    def k(up_h, ee_h, obj_h, zu_h, pu_o, pe_o, idxb, bufs, acc, lsem, ssem):
        sid = lax.axis_index("s")
        cid = lax.axis_index("c")
        wid = sid * NC + cid
        base = wid * per
        zsl = pl.ds(sid * zrows, zrows)
        pltpu.sync_copy(obj_h.at[pl.ds(wid * nck, nck)], idxb)

        def load(c, r):
            pltpu.async_copy(
                src_h.at[pl.ds(pl.multiple_of(base + c * CH, 8), CH)],
                bufs[r], lsem[r])

        def wait_load(r):
            pltpu.make_async_copy(
                src_h.at[pl.ds(pl.multiple_of(base, 8), CH)],
                bufs[r], lsem[r]).wait()

        def wait_scat(r):
            pltpu.make_async_copy(bufs[r], acc.at[idxb.at[0]], ssem[r]).wait()

        for src_h, out_o in ((up_h, pu_o), (ee_h, pe_o)):
            pltpu.sync_copy(zu_h, acc.at[zsl])
            plsc.subcore_barrier()
            for c in range(nck):
                r = c % 2
                if c >= 2:
                    wait_scat(r)
                load(c, r)
                if c >= 1:
                    rp = (c - 1) % 2
                    wait_load(rp)
                    pltpu.async_copy(bufs[rp], acc.at[idxb.at[c - 1]],
                                     ssem[rp], add=True)
            rl = (nck - 1) % 2
            wait_load(rl)
            pltpu.async_copy(bufs[rl], acc.at[idxb.at[nck - 1]],
                             ssem[rl], add=True)
            for r in range(2):
                wait_scat(r)
            plsc.subcore_barrier()
            pltpu.sync_copy(acc.at[zsl], out_o.at[cid, zsl])
            plsc.subcore_barrier()

    return k(up, ee, obj2, z_up)


def _lrelu(x):
    return jnp.maximum(x, 0.01 * x)


def _edge_body(x_r, w1a_r, w1b_r, b1_r, w2_r, b2_r,
               g1_r, g2_r, g3_r, gb_r, h1_r, h2_r, hb_r,
               wst_r, wrt_r, wqrt_r, qb_r, wa_r, up_r, ee_r):
    dot = lambda a, b: jnp.dot(a, b, preferred_element_type=jnp.float32)
    x = x_r[...]
    blk = x.shape[1] * x.shape[2]
    d = x.shape[3]
    hs = x[0].reshape(blk, d)
    rel = x[1].reshape(blk, d)
    tpe = x[2].reshape(blk, d)
    hq = x[3].reshape(blk, d)
    h1 = _lrelu(dot(rel, w1a_r[...]) + dot(tpe, w1b_r[...]) + b1_r[...])
    h2 = _lrelu(dot(h1, w2_r[...]) + b2_r[...])
    hr = h2 + rel
    g = jax.nn.sigmoid(dot(hr, g1_r[...]) + dot(hq, g2_r[...])
                       + dot(hs, g3_r[...]) + gb_r[...])
    d = hs.shape[1]
    upd = g[:, :d]
    rst = g[:, d:]
    cand = jnp.tanh(dot(hr, h1_r[...]) + dot(rst * hs, h2_r[...]) + hb_r[...])
    msg = (1.0 - upd) * hs + upd * cand
    al = _lrelu(dot(hs, wst_r[...]) + dot(hr, wrt_r[...])
                + dot(hq, wqrt_r[...]) + qb_r[...])
    a = jnp.sum(al * wa_r[...], axis=1, keepdims=True)
    e = jnp.exp(a)
    up_r[...] = e * msg
    ee_r[...] = jnp.broadcast_to(e, (e.shape[0], ee_r.shape[1]))


def _edge_compute(x4, w1a, w1b, b1, w2, b2,
                  g1, g2, g3, gb, h1w, h2w, hb, wst, wrt, wqrt, qb, wa):
    nst, nch, ch, d = x4.shape
    blk = 2048
    cpb = blk // ch
    grid = nch // cpb
    ep = nch * ch
    row = lambda n: pl.BlockSpec((blk, n), lambda i: (i, 0))
    full = lambda a: pl.BlockSpec(a.shape, lambda i: (0,) * a.ndim)
    return pl.pallas_call(
        _edge_body,
        grid=(grid,),
        in_specs=[pl.BlockSpec((nst, cpb, ch, d), lambda i: (0, i, 0, 0))]
                 + [full(x) for x in (w1a, w1b, b1, w2, b2, g1, g2, g3,
                                      gb, h1w, h2w, hb, wst, wrt, wqrt, qb, wa)],
        out_specs=(row(d), row(d)),
        out_shape=(jax.ShapeDtypeStruct((ep, d), jnp.float32),
                   jax.ShapeDtypeStruct((ep, d), jnp.float32)),
    )(x4, w1a, w1b, b1, w2, b2, g1, g2, g3, gb,
      h1w, h2w, hb, wst, wrt, wqrt, qb, wa)


def _finish_body(pu_r, pe_r, wh_r, out_r):
    pu = pu_r[...]
    pe = pe_r[...]
    s = pu[0] + pu[1]
    b = pe[0, :, :1] + pe[1, :, :1] + 1e-5
    out_r[...] = jnp.dot(s / b, wh_r[...], preferred_element_type=jnp.float32)


def _finish(pu, pe, wh_t, n):
    d = pu.shape[2]
    de = pe.shape[2]
    blk = 2000
    grid = n // blk
    return pl.pallas_call(
        _finish_body,
        grid=(grid,),
        in_specs=[pl.BlockSpec((NC, blk, d), lambda i: (0, i, 0)),
                  pl.BlockSpec((NC, blk, de), lambda i: (0, i, 0)),
                  pl.BlockSpec(wh_t.shape, lambda i: (0, 0))],
        out_specs=pl.BlockSpec((blk, d), lambda i: (i, 0)),
        out_shape=jax.ShapeDtypeStruct((n, d), jnp.float32),
    )(pu, pe, wh_t)


def kernel(q_sub, q_rel, hidden, edges, n_node, rela_embed, time_pe,
           Ws_W, Wr_W, Wqr_W, Wqr_b, fuse_W1, fuse_b1, fuse_W2, fuse_b2,
           wA, gate_gW, gate_gb, gate_hW, gate_hb, Wh):
    e = edges.shape[0]
    n = hidden.shape[0]
    d = hidden.shape[1]

    step = NW * CH
    ep = ((e + step - 1) // step) * step
    pad = ep - e
    r0 = jnp.concatenate([edges[:, 0], jnp.zeros((pad,), jnp.int32)])
    r2 = jnp.concatenate([edges[:, 2], jnp.zeros((pad,), jnp.int32)])
    sub = jnp.concatenate([edges[:, 4], jnp.zeros((pad,), jnp.int32)])
    t6 = jnp.concatenate([edges[:, 6], jnp.zeros((pad,), jnp.int32)])

    b = q_rel.shape[0]
    bstep = 64 * NW
    bp = ((b + bstep - 1) // bstep) * bstep
    q_rel_pad = jnp.concatenate([q_rel, jnp.zeros((bp - b,), jnp.int32)])

    np_rows = ((n + 1 + NS * 8 - 1) // (NS * 8)) * (NS * 8)
    obj = jnp.concatenate([edges[:, 5], jnp.full((pad,), n, jnp.int32)])

    hq_table = _build_hq_table(rela_embed, q_rel_pad)

    dt = time_pe.shape[1]
    time_pe_pad = jnp.pad(time_pe, ((0, 0), (0, d - dt)))
    nt_rows = bp
    padr = lambda a: jnp.pad(a, ((0, nt_rows - a.shape[0]), (0, 0)))
    tt4 = jnp.concatenate(
        [padr(hidden), padr(rela_embed), padr(time_pe_pad), hq_table], axis=0)
    idx_st = jnp.concatenate([sub, r2, t6, r0])
    out_rows = _gather_staged(tt4, idx_st, 4, nt_rows)
    nchk = ep // CH
    x4 = out_rows.reshape(4, nchk, CH, d)

    w1a = fuse_W1[:, :d].T
    w1b = jnp.pad(fuse_W1[:, d:].T, ((0, d - dt), (0, 0)))
    gt = gate_gW.T
    ht = gate_hW.T
    up, ee = _edge_compute(
        x4,
        w1a, w1b, fuse_b1.reshape(1, -1), fuse_W2.T, fuse_b2.reshape(1, -1),
        gt[:d], gt[d:2 * d], gt[2 * d:], gate_gb.reshape(1, -1),
        ht[:d], ht[d:], gate_hb.reshape(1, -1),
        Ws_W.T, Wr_W.T, Wqr_W.T, Wqr_b.reshape(1, -1), wA)

    zrows = np_rows // NS
    z_up = jnp.zeros((zrows, d), jnp.float32)
    pu, pe = _scatter_segments(up, ee, obj.reshape(-1, CH), np_rows, z_up)
    return _finish(pu, pe, Wh.T, n)


# speedup vs baseline: 1.1766x; 1.0237x over previous
import functools
import math as _math

import jax
import jax.numpy as jnp
import numpy as _np
from jax import lax
from jax.experimental import pallas as pl
from jax.experimental.pallas import tpu as pltpu
from jax.experimental.pallas import tpu_sc as plsc

NC = 2
NS = 16
NW = NC * NS
CH = 128


def _wid():
    return lax.axis_index("s") * NC + lax.axis_index("c")


def _sc_mesh():
    return plsc.VectorSubcoreMesh(core_axis_name="c", subcore_axis_name="s")


def _build_hq_table(rela_embed, q_rel_pad):
    bp = q_rel_pad.shape[0]
    d = rela_embed.shape[1]
    per = bp // NW
    c0 = 64
    nck = per // c0

    @functools.partial(
        pl.kernel,
        mesh=_sc_mesh(),
        out_type=jax.ShapeDtypeStruct((bp, d), jnp.float32),
        scratch_types=[
            pltpu.VMEM((c0,), jnp.int32),
            pltpu.VMEM((c0, d), jnp.float32),
            pltpu.SemaphoreType.DMA,
        ],
    )
    def k(q_h, tab_h, out_h, idx_v, rows_v, sem):
        base = _wid() * per

        def body(j, carry):
            off = pl.multiple_of(base + j * c0, 8)
            pltpu.sync_copy(q_h.at[pl.ds(off, c0)], idx_v)
            pltpu.async_copy(tab_h.at[idx_v], rows_v, sem).wait()
            pltpu.sync_copy(rows_v, out_h.at[pl.ds(off, c0)])
            return carry

        lax.fori_loop(0, nck, body, 0)

    return k(q_rel_pad, rela_embed)


def _gather_staged(tt4, idx_st, n_tab, rows_per_tab):
    total = idx_st.shape[0]
    d = tt4.shape[1]
    ep = total // n_tab
    per_t = ep // CH // NW
    G = next(g for g in (20, 16, 10, 8, 5, 4, 2, 1) if per_t % g == 0)
    R = 2
    ngr = per_t // G
    srows = rows_per_tab // NS

    @functools.partial(
        pl.kernel,
        mesh=_sc_mesh(),
        out_type=jax.ShapeDtypeStruct((total, d), jnp.float32),
        scratch_types=[
            pltpu.VMEM_SHARED((rows_per_tab, d), jnp.float32),
            pltpu.VMEM((G * CH,), jnp.int32),
            [pltpu.VMEM((CH, d), jnp.float32)] * R,
            [pltpu.SemaphoreType.DMA] * R,
            [pltpu.SemaphoreType.DMA] * R,
        ],
    )
    def k(tt_h, idx_h, out_h, stab, idxb, bufs, gsem, ssem):
        sid = lax.axis_index("s")
        base = _wid() * per_t * CH
        ssl = pl.ds(sid * srows, srows)

        for t in range(n_tab):
            pltpu.sync_copy(tt_h.at[pl.ds(t * rows_per_tab + sid * srows,
                                          srows)], stab.at[ssl])
            plsc.subcore_barrier()

            def group(g, carry):
                goff = pl.multiple_of(t * ep + base + g * (G * CH), 8)
                pltpu.sync_copy(idx_h.at[pl.ds(goff, G * CH)], idxb)
                pltpu.async_copy(stab.at[idxb.at[pl.ds(0, CH)]],
                                 bufs[0], gsem[0])
                for c in range(G):
                    r = c % R
                    if c + 1 < G:
                        rr = (c + 1) % R
                        if c - 1 >= 0:
                            pltpu.make_async_copy(
                                bufs[rr],
                                out_h.at[pl.ds(goff + (c - 1) * CH, CH)],
                                ssem[rr]).wait()
                        pltpu.async_copy(
                            stab.at[idxb.at[pl.ds((c + 1) * CH, CH)]],
                            bufs[rr], gsem[rr])
                    pltpu.make_async_copy(stab.at[idxb.at[pl.ds(c * CH, CH)]],
                                          bufs[r], gsem[r]).wait()
                    pltpu.async_copy(bufs[r],
                                     out_h.at[pl.ds(goff + c * CH, CH)],
                                     ssem[r])
                for r in range(R):
                    pltpu.make_async_copy(bufs[r], out_h.at[pl.ds(goff, CH)],
                                          ssem[r]).wait()
                return carry

            lax.fori_loop(0, ngr, group, 0)
            plsc.subcore_barrier()

    return k(tt4, idx_st)


def _scatter_segments(up, ee, obj2, np_rows, z_up):
    nchk = obj2.shape[0]
    ep = nchk * CH
    per = ep // NW
    nck = per // CH
    d = up.shape[1]
    zrows = np_rows // NS

    @functools.partial(
        pl.kernel,
        mesh=_sc_mesh(),
        out_type=(
            jax.ShapeDtypeStruct((NC, np_rows, d), jnp.float32),
            jax.ShapeDtypeStruct((NC, np_rows, d), jnp.float32),
        ),
        scratch_types=[
            pltpu.VMEM((nck, CH), jnp.int32),
            [pltpu.VMEM((CH, d), jnp.float32)] * 2,
            pltpu.VMEM_SHARED((np_rows, d), jnp.float32),
            [pltpu.SemaphoreType.DMA] * 2,
            [pltpu.SemaphoreType.DMA] * 2,
        ],
    )
    def k(up_h, ee_h, obj_h, zu_h, pu_o, pe_o, idxb, bufs, acc, lsem, ssem):
        sid = lax.axis_index("s")
        cid = lax.axis_index("c")
        wid = sid * NC + cid
        base = wid * per
        zsl = pl.ds(sid * zrows, zrows)
        pltpu.sync_copy(obj_h.at[pl.ds(wid * nck, nck)], idxb)

        def load(c, r):
            pltpu.async_copy(
                src_h.at[pl.ds(pl.multiple_of(base + c * CH, 8), CH)],
                bufs[r], lsem[r])

        def wait_load(r):
            pltpu.make_async_copy(
                src_h.at[pl.ds(pl.multiple_of(base, 8), CH)],
                bufs[r], lsem[r]).wait()

        def wait_scat(r):
            pltpu.make_async_copy(bufs[r], acc.at[idxb.at[0]], ssem[r]).wait()

        for src_h, out_o in ((up_h, pu_o), (ee_h, pe_o)):
            pltpu.sync_copy(zu_h, acc.at[zsl])
            plsc.subcore_barrier()
            for c in range(nck):
                r = c % 2
                if c >= 2:
                    wait_scat(r)
                load(c, r)
                if c >= 1:
                    rp = (c - 1) % 2
                    wait_load(rp)
                    pltpu.async_copy(bufs[rp], acc.at[idxb.at[c - 1]],
                                     ssem[rp], add=True)
            rl = (nck - 1) % 2
            wait_load(rl)
            pltpu.async_copy(bufs[rl], acc.at[idxb.at[nck - 1]],
                             ssem[rl], add=True)
            for r in range(2):
                wait_scat(r)
            plsc.subcore_barrier()
            pltpu.sync_copy(acc.at[zsl], out_o.at[cid, zsl])
            plsc.subcore_barrier()

    return k(up, ee, obj2, z_up)


def _lrelu(x):
    return jnp.maximum(x, 0.01 * x)


def _edge_body(x_r, w1a_r, w1b_r, b1_r, w2_r, b2_r,
               g1_r, g2_r, g3_r, gb_r, h1_r, h2_r, hb_r,
               wst_r, wrt_r, wqrt_r, qb_r, wa_r, up_r, ee_r):
    dot = lambda a, b: jnp.dot(a, b, preferred_element_type=jnp.float32)
    x = x_r[...]
    blk = x.shape[1] * x.shape[2]
    d = x.shape[3]
    hs = x[0].reshape(blk, d)
    rel = x[1].reshape(blk, d)
    tpe = x[2].reshape(blk, d)
    hq = x[3].reshape(blk, d)
    h1 = _lrelu(dot(rel, w1a_r[...]) + dot(tpe, w1b_r[...]) + b1_r[...])
    h2 = _lrelu(dot(h1, w2_r[...]) + b2_r[...])
    hr = h2 + rel
    g = jax.nn.sigmoid(dot(hr, g1_r[...]) + dot(hq, g2_r[...])
                       + dot(hs, g3_r[...]) + gb_r[...])
    d = hs.shape[1]
    upd = g[:, :d]
    rst = g[:, d:]
    cand = jnp.tanh(dot(hr, h1_r[...]) + dot(rst * hs, h2_r[...]) + hb_r[...])
    msg = (1.0 - upd) * hs + upd * cand
    al = _lrelu(dot(hs, wst_r[...]) + dot(hr, wrt_r[...])
                + dot(hq, wqrt_r[...]) + qb_r[...])
    a = jnp.sum(al * wa_r[...], axis=1, keepdims=True)
    e = jnp.exp(a)
    up_r[...] = e * msg
    ee_r[...] = jnp.broadcast_to(e, (e.shape[0], ee_r.shape[1]))


def _edge_compute(x4, w1a, w1b, b1, w2, b2,
                  g1, g2, g3, gb, h1w, h2w, hb, wst, wrt, wqrt, qb, wa):
    nst, nch, ch, d = x4.shape
    blk = 4096
    cpb = blk // ch
    grid = nch // cpb
    ep = nch * ch
    row = lambda n: pl.BlockSpec((blk, n), lambda i: (i, 0))
    full = lambda a: pl.BlockSpec(a.shape, lambda i: (0,) * a.ndim)
    return pl.pallas_call(
        _edge_body,
        grid=(grid,),
        in_specs=[pl.BlockSpec((nst, cpb, ch, d), lambda i: (0, i, 0, 0))]
                 + [full(x) for x in (w1a, w1b, b1, w2, b2, g1, g2, g3,
                                      gb, h1w, h2w, hb, wst, wrt, wqrt, qb, wa)],
        out_specs=(row(d), row(d)),
        out_shape=(jax.ShapeDtypeStruct((ep, d), jnp.float32),
                   jax.ShapeDtypeStruct((ep, d), jnp.float32)),
    )(x4, w1a, w1b, b1, w2, b2, g1, g2, g3, gb,
      h1w, h2w, hb, wst, wrt, wqrt, qb, wa)


def _finish_body(pu_r, pe_r, wh_r, out_r):
    pu = pu_r[...]
    pe = pe_r[...]
    s = pu[0] + pu[1]
    b = pe[0, :, :1] + pe[1, :, :1] + 1e-5
    out_r[...] = jnp.dot(s / b, wh_r[...], preferred_element_type=jnp.float32)


def _finish(pu, pe, wh_t, n):
    d = pu.shape[2]
    de = pe.shape[2]
    blk = 2000
    grid = n // blk
    return pl.pallas_call(
        _finish_body,
        grid=(grid,),
        in_specs=[pl.BlockSpec((NC, blk, d), lambda i: (0, i, 0)),
                  pl.BlockSpec((NC, blk, de), lambda i: (0, i, 0)),
                  pl.BlockSpec(wh_t.shape, lambda i: (0, 0))],
        out_specs=pl.BlockSpec((blk, d), lambda i: (i, 0)),
        out_shape=jax.ShapeDtypeStruct((n, d), jnp.float32),
    )(pu, pe, wh_t)


def kernel(q_sub, q_rel, hidden, edges, n_node, rela_embed, time_pe,
           Ws_W, Wr_W, Wqr_W, Wqr_b, fuse_W1, fuse_b1, fuse_W2, fuse_b2,
           wA, gate_gW, gate_gb, gate_hW, gate_hb, Wh):
    e = edges.shape[0]
    n = hidden.shape[0]
    d = hidden.shape[1]

    step = NW * CH
    ep = ((e + step - 1) // step) * step
    pad = ep - e
    r0 = jnp.concatenate([edges[:, 0], jnp.zeros((pad,), jnp.int32)])
    r2 = jnp.concatenate([edges[:, 2], jnp.zeros((pad,), jnp.int32)])
    sub = jnp.concatenate([edges[:, 4], jnp.zeros((pad,), jnp.int32)])
    t6 = jnp.concatenate([edges[:, 6], jnp.zeros((pad,), jnp.int32)])

    b = q_rel.shape[0]
    bstep = 64 * NW
    bp = ((b + bstep - 1) // bstep) * bstep
    q_rel_pad = jnp.concatenate([q_rel, jnp.zeros((bp - b,), jnp.int32)])

    np_rows = ((n + 1 + NS * 8 - 1) // (NS * 8)) * (NS * 8)
    obj = jnp.concatenate([edges[:, 5], jnp.full((pad,), n, jnp.int32)])

    hq_table = _build_hq_table(rela_embed, q_rel_pad)

    dt = time_pe.shape[1]
    time_pe_pad = jnp.pad(time_pe, ((0, 0), (0, d - dt)))
    nt_rows = bp
    padr = lambda a: jnp.pad(a, ((0, nt_rows - a.shape[0]), (0, 0)))
    tt4 = jnp.concatenate(
        [padr(hidden), padr(rela_embed), padr(time_pe_pad), hq_table], axis=0)
    idx_st = jnp.concatenate([sub, r2, t6, r0])
    out_rows = _gather_staged(tt4, idx_st, 4, nt_rows)
    nchk = ep // CH
    x4 = out_rows.reshape(4, nchk, CH, d)

    w1a = fuse_W1[:, :d].T
    w1b = jnp.pad(fuse_W1[:, d:].T, ((0, d - dt), (0, 0)))
    gt = gate_gW.T
    ht = gate_hW.T
    up, ee = _edge_compute(
        x4,
        w1a, w1b, fuse_b1.reshape(1, -1), fuse_W2.T, fuse_b2.reshape(1, -1),
        gt[:d], gt[d:2 * d], gt[2 * d:], gate_gb.reshape(1, -1),
        ht[:d], ht[d:], gate_hb.reshape(1, -1),
        Ws_W.T, Wr_W.T, Wqr_W.T, Wqr_b.reshape(1, -1), wA)

    zrows = np_rows // NS
    z_up = jnp.zeros((zrows, d), jnp.float32)
    pu, pe = _scatter_segments(up, ee, obj.reshape(-1, CH), np_rows, z_up)
    return _finish(pu, pe, Wh.T, n)
